# baseline (speedup 1.0000x reference)
import jax
import jax.numpy as jnp
from jax.experimental import pallas as pl


def _gcn_j(x, ei, ew, W, b, N):
    src, dst = ei[0], ei[1]
    loop = jnp.arange(N, dtype=src.dtype)
    src = jnp.concatenate([src, loop])
    dst = jnp.concatenate([dst, loop])
    if ew is None:
        ew_full = jnp.ones(src.shape[0], jnp.float32)
    else:
        ew_full = jnp.concatenate([ew, jnp.ones(N, jnp.float32)])
    deg = jax.ops.segment_sum(ew_full, dst, num_segments=N)
    dinv = jnp.where(deg > 0, 1.0 / jnp.sqrt(jnp.where(deg > 0, deg, 1.0)), 0.0)
    norm = dinv[src] * ew_full * dinv[dst]
    h = x @ W
    out = jax.ops.segment_sum(h[src] * norm[:, None], dst, num_segments=N) + b
    return out


def _gat_j(x, ei, W, a_src, a_dst, b, N):
    src, dst = ei[0], ei[1]
    loop = jnp.arange(N, dtype=src.dtype)
    src = jnp.concatenate([src, loop])
    dst = jnp.concatenate([dst, loop])
    h = x @ W
    al_s = h @ a_src
    al_d = h @ a_dst
    e = jax.nn.leaky_relu(al_s[src] + al_d[dst], 0.2)
    emax = jax.ops.segment_max(e, dst, num_segments=N)
    ee = jnp.exp(e - emax[dst])
    denom = jax.ops.segment_sum(ee, dst, num_segments=N)
    alpha = ee / (denom[dst] + 1e-16)
    out = jax.ops.segment_sum(h[src] * alpha[:, None], dst, num_segments=N) + b
    return out


def _mean_pool_j(x, batch, nb):
    s = jax.ops.segment_sum(x, batch, num_segments=nb)
    cnt = jax.ops.segment_sum(jnp.ones((x.shape[0],), jnp.float32), batch, num_segments=nb)
    return s / jnp.clip(cnt, 1.0)[:, None]


def _ident_body(x_ref, o_ref):
    o_ref[...] = x_ref[...]


def kernel(x1, edge_index1, edge_attr1, batch1, x3, edge_index3, ami_dis, ami_batch, ami_dis_li, w1, b1, w2, b2, Wc1, bc1, Wg, a_src, a_dst, bg, Wc4, bc4, Wc5, bc5, Wfc1, bfc1, Wfc2, bfc2, Wout, bout):
    N1, N3, B = x1.shape[0], x3.shape[0], 256
    ami = _gcn_j(x3, edge_index3, ami_dis, Wc4, bc4, N3)
    ami = jax.nn.relu(ami)
    ami = jnp.concatenate([ami, ami_dis_li[:, None]], axis=1)
    ami = _gcn_j(ami, edge_index3, ami_dis, Wc5, bc5, N3)
    ami = jax.nn.relu(ami)
    ami = _mean_pool_j(ami, ami_batch, B)
    h = jax.nn.relu(x1 @ w1 + b1)
    h = jax.nn.relu(h @ w2 + b2)
    h = jax.nn.relu(_gcn_j(h, edge_index1, None, Wc1, bc1, N1))
    h = jax.nn.relu(_gat_j(h, edge_index1, Wg, a_src, a_dst, bg, N1))
    h = _mean_pool_j(h, batch1, B)
    x = jnp.concatenate([ami, h], axis=1)
    h2 = x @ Wfc1 + bfc1
    h2 = pl.pallas_call(
        _ident_body,
        out_shape=jax.ShapeDtypeStruct(h2.shape, h2.dtype),
    )(h2)
    x = jax.nn.relu(h2)
    x = jax.nn.relu(x @ Wfc2 + bfc2)
    out = x @ Wout + bout
    return (out, h2)


# speedup vs baseline: 12.2083x; 12.2083x over previous
import jax
import jax.numpy as jnp
from jax import lax
from jax.experimental import pallas as pl
from jax.experimental.pallas import tpu as pltpu
from jax.experimental.pallas import tpu_sc as plsc

N1, E1 = 50000, 800000
N3, E3 = 10000, 40000
B = 256

CHUNK = 2048
EP1 = 851968
EP3 = 65536
NP1 = 51200
NP3 = 10240
SP1 = 53248
SP3 = 12288

_MESH = dict(core_axis_name="c", subcore_axis_name="s",
             num_cores=2, num_subcores=16)


def _make_sc_segsum(Npad, Epad, has_vals):
    nch = Epad // CHUNK
    nchc = nch // 2
    tpc = nchc // 16
    nz = Npad // CHUNK

    def body(*refs):
        if has_vals:
            dst2d, vals2d, out, spacc, dbuf, vbuf, didx, dval, zbuf = refs
        else:
            dst2d, out, spacc, dbuf, vbuf, didx, dval, zbuf = refs
        c = lax.axis_index("c")
        s = lax.axis_index("s")

        def zb(i, _):
            zbuf[pl.ds(i * 16, 16)] = jnp.zeros((16,), jnp.float32)
            return 0
        lax.fori_loop(0, CHUNK // 16, zb, 0)
        if not has_vals:
            def ob(i, _):
                vbuf[i // 8, pl.ds((i % 8) * 16, 16)] = jnp.ones((16,), jnp.float32)
                return 0
            lax.fori_loop(0, 128, ob, 0)

        for t in range(-(-nz // 16)):
            j = s + 16 * t
            @pl.when(j < nz)
            def _():
                pltpu.sync_copy(zbuf, spacc.at[pl.ds(j * CHUNK, CHUNK)])
        plsc.subcore_barrier()

        def chunk(t, _):
            j = (c * nchc + s + 16 * t) * 16
            pltpu.sync_copy(dst2d.at[pl.ds(j, 16)], dbuf)
            if has_vals:
                pltpu.sync_copy(vals2d.at[pl.ds(j, 16)], vbuf)

            def row(r, _):
                def mv(k, _):
                    didx[pl.ds(k * 16, 16)] = dbuf[r, pl.ds(k * 16, 16)]
                    dval[pl.ds(k * 16, 16)] = vbuf[r, pl.ds(k * 16, 16)]
                    return 0
                lax.fori_loop(0, 8, mv, 0)
                pltpu.sync_copy(dval, spacc.at[didx], add=True)
                return 0
            lax.fori_loop(0, 16, row, 0)
            return 0
        lax.fori_loop(0, tpc, chunk, 0)
        plsc.subcore_barrier()

        for t in range(-(-nz // 16)):
            j = s + 16 * t
            @pl.when(j < nz)
            def _():
                pltpu.sync_copy(spacc.at[pl.ds(j * CHUNK, CHUNK)],
                                out.at[c, pl.ds(j * CHUNK, CHUNK)])

    return pl.kernel(
        body,
        out_type=jax.ShapeDtypeStruct((2, Npad), jnp.float32),
        mesh=plsc.VectorSubcoreMesh(**_MESH),
        compiler_params=pltpu.CompilerParams(needs_layout_passes=False),
        scratch_types=[
            pltpu.VMEM_SHARED((Npad,), jnp.float32),
            pltpu.VMEM((16, 128), jnp.int32),
            pltpu.VMEM((16, 128), jnp.float32),
            pltpu.VMEM((128,), jnp.int32),
            pltpu.VMEM((128,), jnp.float32),
            pltpu.VMEM((CHUNK,), jnp.float32),
        ],
    )


def _make_sc_rowagg(N, Epad, D, nq, has_w, G=128):
    Nq = N // nq
    NqP = Nq + 128
    nz = NqP // 128
    nch = Epad // CHUNK
    tpc = nch // 16
    npass = nq // 2
    CAPG = G + CHUNK + 16

    def body(*refs):
        if has_w:
            (table, src2d, dst2d, w2d, out, spacc, csrc, cdst, cw,
             rowbuf, sidx, didx, sw, sbuf, dbuf, wbuf, sem) = refs
        else:
            (table, src2d, dst2d, out, spacc, csrc, cdst,
             rowbuf, sidx, didx, sbuf, dbuf, sem) = refs
        c = lax.axis_index("c")
        s = lax.axis_index("s")

        for p in range(npass):
            q = c * npass + p
            lo = q * Nq

            def zr(i, _):
                rowbuf[i // (D // 16), pl.ds((i % (D // 16)) * 16, 16)] = (
                    jnp.zeros((16,), jnp.float32))
                return 0
            lax.fori_loop(0, 128 * D // 16, zr, 0)

            for t in range(-(-nz // 16)):
                j = s + 16 * t
                @pl.when(j < nz)
                def _():
                    pltpu.sync_copy(rowbuf, spacc.at[pl.ds(j * 128, 128)])
            plsc.subcore_barrier()

            def drain(i, _):
                base = i * G

                def stg(m, _):
                    sidx[pl.ds(m * 16, 16)] = csrc[pl.ds(base + m * 16, 16)]
                    didx[pl.ds(m * 16, 16)] = cdst[pl.ds(base + m * 16, 16)]
                    if has_w:
                        sw[pl.ds(m * 16, 16)] = cw[pl.ds(base + m * 16, 16)]
                    return 0
                lax.fori_loop(0, G // 16, stg, 0)

                pltpu.async_copy(table.at[sidx], rowbuf, sem).wait()

                if has_w:
                    def scale(r, _):
                        w = sw[pl.ds(r, 16)][0]
                        for k in range(D // 16):
                            rowbuf[r, pl.ds(k * 16, 16)] = (
                                rowbuf[r, pl.ds(k * 16, 16)]
                                * jnp.full((16,), w, jnp.float32))
                        return 0
                    lax.fori_loop(0, G, scale, 0)

                pltpu.sync_copy(rowbuf, spacc.at[didx], add=True)
                return 0

            def chunk(t, na):
                j = (s + 16 * t) * 16
                pltpu.sync_copy(src2d.at[pl.ds(j, 16)], sbuf)
                pltpu.sync_copy(dst2d.at[pl.ds(j, 16)], dbuf)
                if has_w:
                    pltpu.sync_copy(w2d.at[pl.ds(j, 16)], wbuf)

                def vec(v, na):
                    r, k = v // 8, (v % 8) * 16
                    dvec = dbuf[r, pl.ds(k, 16)]
                    svec = sbuf[r, pl.ds(k, 16)]
                    d = dvec - lo
                    msk = (d >= 0) & (d < Nq)
                    plsc.store_compressed(csrc.at[pl.ds(na, 16)], svec, mask=msk)
                    plsc.store_compressed(cdst.at[pl.ds(na, 16)], d, mask=msk)
                    if has_w:
                        wv = wbuf[r, pl.ds(k, 16)]
                        plsc.store_compressed(cw.at[pl.ds(na, 16)], wv, mask=msk)
                    cnt = jnp.sum(msk.astype(jnp.int32), axis=0)
                    return na + cnt
                na = lax.fori_loop(0, 128, vec, na)

                done = na // G
                lax.fori_loop(0, done, drain, 0)

                @pl.when(done > 0)
                def _():
                    def cpy(m, _):
                        csrc[pl.ds(m * 16, 16)] = csrc[pl.ds(done * G + m * 16, 16)]
                        cdst[pl.ds(m * 16, 16)] = cdst[pl.ds(done * G + m * 16, 16)]
                        if has_w:
                            cw[pl.ds(m * 16, 16)] = cw[pl.ds(done * G + m * 16, 16)]
                        return 0
                    lax.fori_loop(0, G // 16, cpy, 0)
                return na - done * G
            na = lax.fori_loop(0, tpc, chunk, 0)

            for k in range(G // 16):
                csrc[pl.ds(na + k * 16, 16)] = jnp.zeros((16,), jnp.int32)
                cdst[pl.ds(na + k * 16, 16)] = jnp.full((16,), Nq, jnp.int32)
                if has_w:
                    cw[pl.ds(na + k * 16, 16)] = jnp.zeros((16,), jnp.float32)

            @pl.when(na > 0)
            def _():
                drain(0, 0)
            plsc.subcore_barrier()

            for t in range(-(-nz // 16)):
                j = s + 16 * t
                @pl.when(j < nz)
                def _():
                    pltpu.sync_copy(spacc.at[pl.ds(j * 128, 128)],
                                    out.at[q, pl.ds(j * 128, 128)])
            if npass > 1:
                plsc.subcore_barrier()

    scratch = [
        pltpu.VMEM_SHARED((NqP, D), jnp.float32),
        pltpu.VMEM((CAPG,), jnp.int32),
        pltpu.VMEM((CAPG,), jnp.int32),
    ]
    if has_w:
        scratch.append(pltpu.VMEM((CAPG,), jnp.float32))
    scratch += [
        pltpu.VMEM((128, D), jnp.float32),
        pltpu.VMEM((128,), jnp.int32),
        pltpu.VMEM((128,), jnp.int32),
    ]
    if has_w:
        scratch.append(pltpu.VMEM((144,), jnp.float32))
    scratch += [
        pltpu.VMEM((16, 128), jnp.int32),
        pltpu.VMEM((16, 128), jnp.int32),
    ]
    if has_w:
        scratch.append(pltpu.VMEM((16, 128), jnp.float32))
    scratch.append(pltpu.SemaphoreType.DMA)

    kern = pl.kernel(
        body,
        out_type=jax.ShapeDtypeStruct((nq, NqP, D), jnp.float32),
        mesh=plsc.VectorSubcoreMesh(**_MESH),
        compiler_params=pltpu.CompilerParams(needs_layout_passes=False),
        scratch_types=scratch,
    )
    return kern, NqP


def _make_sc_el(N, Epad):
    tpt = (Epad // CHUNK) // 32

    def body(als_h, ald_h, src2d, dst2d, elP, alsv, aldv, sbuf, dbuf, ebuf):
        c = lax.axis_index("c")
        s = lax.axis_index("s")
        wid = s * 2 + c
        pltpu.sync_copy(als_h, alsv)
        pltpu.sync_copy(ald_h, aldv)

        def chunk(t, _):
            j = (wid + 32 * t) * 16
            pltpu.sync_copy(src2d.at[pl.ds(j, 16)], sbuf)
            pltpu.sync_copy(dst2d.at[pl.ds(j, 16)], dbuf)

            def vec(v, _):
                r, k = v // 8, (v % 8) * 16
                svec = jnp.minimum(sbuf[r, pl.ds(k, 16)], N - 1)
                dvec = jnp.minimum(dbuf[r, pl.ds(k, 16)], N - 1)
                x = plsc.load_gather(alsv, [svec]) + plsc.load_gather(aldv, [dvec])
                ebuf[r, pl.ds(k, 16)] = jnp.where(x >= 0, x, 0.2 * x)
                return 0
            lax.fori_loop(0, 128, vec, 0)
            pltpu.sync_copy(ebuf, elP.at[pl.ds(j, 16)])
            return 0
        lax.fori_loop(0, tpt, chunk, 0)

    return pl.kernel(
        body,
        out_type=jax.ShapeDtypeStruct((Epad // 128, 128), jnp.float32),
        mesh=plsc.VectorSubcoreMesh(**_MESH),
        compiler_params=pltpu.CompilerParams(needs_layout_passes=False),
        scratch_types=[
            pltpu.VMEM((N,), jnp.float32),
            pltpu.VMEM((N,), jnp.float32),
            pltpu.VMEM((16, 128), jnp.int32),
            pltpu.VMEM((16, 128), jnp.int32),
            pltpu.VMEM((16, 128), jnp.float32),
        ],
    )


def _make_sc_max(N, Epad):
    tpt = (Epad // CHUNK) // 32

    def body(dst2d, elP, out, macc, dbuf, ebuf):
        c = lax.axis_index("c")
        s = lax.axis_index("s")
        wid = s * 2 + c

        def ini(i, _):
            macc[pl.ds(i * 16, 16)] = jnp.full((16,), -3e38, jnp.float32)
            return 0
        lax.fori_loop(0, (N + 16) // 16, ini, 0)

        def chunk(t, _):
            j = (wid + 32 * t) * 16
            pltpu.sync_copy(dst2d.at[pl.ds(j, 16)], dbuf)
            pltpu.sync_copy(elP.at[pl.ds(j, 16)], ebuf)

            def vec(v, _):
                r, k = v // 8, (v % 8) * 16
                dvec = dbuf[r, pl.ds(k, 16)]
                evec = ebuf[r, pl.ds(k, 16)]
                valid = dvec < N
                dc = jnp.minimum(dvec, N)

                def cond(it):
                    cur = plsc.load_gather(macc, [dc])
                    return jnp.any(valid & (evec > cur)) & (it < 17)

                def bd(it):
                    cur = plsc.load_gather(macc, [dc])
                    m = valid & (evec > cur)
                    plsc.store_scatter(macc, [dc], evec, mask=m)
                    return it + 1
                lax.while_loop(cond, bd, 0)
                return 0
            lax.fori_loop(0, 128, vec, 0)
            return 0
        lax.fori_loop(0, tpt, chunk, 0)
        pltpu.sync_copy(macc.at[pl.ds(0, N)], out.at[wid])

    return pl.kernel(
        body,
        out_type=jax.ShapeDtypeStruct((32, N), jnp.float32),
        mesh=plsc.VectorSubcoreMesh(**_MESH),
        compiler_params=pltpu.CompilerParams(needs_layout_passes=False),
        scratch_types=[
            pltpu.VMEM((N + 16,), jnp.float32),
            pltpu.VMEM((16, 128), jnp.int32),
            pltpu.VMEM((16, 128), jnp.float32),
        ],
    )


def _make_sc_den(N, Npad, Epad):
    nch = Epad // CHUNK
    nchc = nch // 2
    tpc = nchc // 16
    nz = Npad // CHUNK

    def body(dst2d, elP, m_h, denp, eeP, spden, mv, dbuf, ebuf, eebuf,
             didx, dval, zbuf):
        c = lax.axis_index("c")
        s = lax.axis_index("s")
        pltpu.sync_copy(m_h, mv)

        def zb(i, _):
            zbuf[pl.ds(i * 16, 16)] = jnp.zeros((16,), jnp.float32)
            return 0
        lax.fori_loop(0, CHUNK // 16, zb, 0)
        for t in range(-(-nz // 16)):
            j = s + 16 * t
            @pl.when(j < nz)
            def _():
                pltpu.sync_copy(zbuf, spden.at[pl.ds(j * CHUNK, CHUNK)])
        plsc.subcore_barrier()

        def chunk(t, _):
            j = (c * nchc + s + 16 * t) * 16
            pltpu.sync_copy(dst2d.at[pl.ds(j, 16)], dbuf)
            pltpu.sync_copy(elP.at[pl.ds(j, 16)], ebuf)

            def vec(v, _):
                r, k = v // 8, (v % 8) * 16
                dvec = jnp.minimum(dbuf[r, pl.ds(k, 16)], N - 1)
                mg = plsc.load_gather(mv, [dvec])
                eebuf[r, pl.ds(k, 16)] = jnp.exp(ebuf[r, pl.ds(k, 16)] - mg)
                return 0
            lax.fori_loop(0, 128, vec, 0)
            pltpu.sync_copy(eebuf, eeP.at[pl.ds(j, 16)])

            def row(r, _):
                def mv2(k, _):
                    didx[pl.ds(k * 16, 16)] = dbuf[r, pl.ds(k * 16, 16)]
                    dval[pl.ds(k * 16, 16)] = eebuf[r, pl.ds(k * 16, 16)]
                    return 0
                lax.fori_loop(0, 8, mv2, 0)
                pltpu.sync_copy(dval, spden.at[didx], add=True)
                return 0
            lax.fori_loop(0, 16, row, 0)
            return 0
        lax.fori_loop(0, tpc, chunk, 0)
        plsc.subcore_barrier()

        for t in range(-(-nz // 16)):
            j = s + 16 * t
            @pl.when(j < nz)
            def _():
                pltpu.sync_copy(spden.at[pl.ds(j * CHUNK, CHUNK)],
                                denp.at[c, pl.ds(j * CHUNK, CHUNK)])

    return pl.kernel(
        body,
        out_type=(jax.ShapeDtypeStruct((2, Npad), jnp.float32),
                  jax.ShapeDtypeStruct((Epad // 128, 128), jnp.float32)),
        mesh=plsc.VectorSubcoreMesh(**_MESH),
        compiler_params=pltpu.CompilerParams(needs_layout_passes=False),
        scratch_types=[
            pltpu.VMEM_SHARED((Npad,), jnp.float32),
            pltpu.VMEM((N,), jnp.float32),
            pltpu.VMEM((16, 128), jnp.int32),
            pltpu.VMEM((16, 128), jnp.float32),
            pltpu.VMEM((16, 128), jnp.float32),
            pltpu.VMEM((128,), jnp.int32),
            pltpu.VMEM((128,), jnp.float32),
            pltpu.VMEM((CHUNK,), jnp.float32),
        ],
    )


def _m1_call(x1p, w1, b1, w2, b2, degpT):
    BM = 2048
    grid = NP1 // BM

    def body(x_r, w1_r, b1_r, w2_r, b2_r, degp_r, g1p_r, dinv_r):
        deg = 1.0 + degp_r[:, 0] + degp_r[:, 1]
        dinv = lax.rsqrt(deg)
        h0 = jax.nn.relu(jnp.dot(x_r[...], w1_r[...]) + b1_r[...][None, :])
        h1 = jax.nn.relu(jnp.dot(h0, w2_r[...]) + b2_r[...][None, :])
        g1 = h1 * dinv[:, None]
        g1p_r[...] = jnp.concatenate(
            [g1, jnp.zeros((BM, 64), jnp.float32)], axis=1)
        dinv_r[...] = dinv

    return pl.pallas_call(
        body,
        grid=(grid,),
        in_specs=[
            pl.BlockSpec((BM, 52), lambda i: (i, 0)),
            pl.BlockSpec((52, 128), lambda i: (0, 0)),
            pl.BlockSpec((128,), lambda i: (0,)),
            pl.BlockSpec((128, 64), lambda i: (0, 0)),
            pl.BlockSpec((64,), lambda i: (0,)),
            pl.BlockSpec((BM, 2), lambda i: (i, 0)),
        ],
        out_specs=[
            pl.BlockSpec((BM, 128), lambda i: (i, 0)),
            pl.BlockSpec((BM,), lambda i: (i,)),
        ],
        out_shape=[
            jax.ShapeDtypeStruct((NP1, 128), jnp.float32),
            jax.ShapeDtypeStruct((NP1,), jnp.float32),
        ],
    )(x1p, w1, b1, w2, b2, degpT)


def _m2_call(x3p, Wc4p, degpT):
    BM = 1024
    grid = NP3 // BM

    def body(x_r, w_r, degp_r, *outs):
        dinv_r = outs[8]
        deg = 1.0 + degp_r[:, 0] + degp_r[:, 1]
        dinv = lax.rsqrt(deg)
        p1 = jnp.dot(x_r[...], w_r[...]) * dinv[:, None]
        for k in range(8):
            outs[k][...] = p1[:, k * 128:(k + 1) * 128]
        dinv_r[...] = dinv

    outs = [jax.ShapeDtypeStruct((NP3, 128), jnp.float32)] * 8 + [
        jax.ShapeDtypeStruct((NP3,), jnp.float32)]
    return pl.pallas_call(
        body,
        grid=(grid,),
        in_specs=[
            pl.BlockSpec((BM, 1900), lambda i: (i, 0)),
            pl.BlockSpec((1900, 1024), lambda i: (0, 0)),
            pl.BlockSpec((BM, 2), lambda i: (i, 0)),
        ],
        out_specs=[pl.BlockSpec((BM, 128), lambda i: (i, 0))] * 8 + [
            pl.BlockSpec((BM,), lambda i: (i,))],
        out_shape=outs,
    )(x3p, Wc4p, degpT)


def _m3_call(agg1, g1p, dinv1, Wc1p, bc1, Wg, a_src, a_dst):
    BM = 2048
    grid = NP1 // BM

    def body(agg_r, g1p_r, dinv_r, w_r, b_r, wg_r, asr_r, adr_r,
             h2d_r, als_r, ald_r, els_r):
        dinv = dinv_r[...]
        full = (agg_r[...] + g1p_r[...]) * dinv[:, None]
        h2d = jax.nn.relu(jnp.dot(full, w_r[...]) + b_r[...][None, :])
        ws = jnp.dot(wg_r[...], asr_r[...])
        wd = jnp.dot(wg_r[...], adr_r[...])
        al_s = jnp.sum(h2d * ws[None, :], axis=1)
        al_d = jnp.sum(h2d * wd[None, :], axis=1)
        x = al_s + al_d
        h2d_r[...] = h2d
        als_r[...] = al_s
        ald_r[...] = al_d
        els_r[...] = jnp.where(x >= 0, x, 0.2 * x)

    return pl.pallas_call(
        body,
        grid=(grid,),
        in_specs=[
            pl.BlockSpec((BM, 128), lambda i: (i, 0)),
            pl.BlockSpec((BM, 128), lambda i: (i, 0)),
            pl.BlockSpec((BM,), lambda i: (i,)),
            pl.BlockSpec((128, 128), lambda i: (0, 0)),
            pl.BlockSpec((128,), lambda i: (0,)),
            pl.BlockSpec((128, 256), lambda i: (0, 0)),
            pl.BlockSpec((256,), lambda i: (0,)),
            pl.BlockSpec((256,), lambda i: (0,)),
        ],
        out_specs=[
            pl.BlockSpec((BM, 128), lambda i: (i, 0)),
            pl.BlockSpec((BM,), lambda i: (i,)),
            pl.BlockSpec((BM,), lambda i: (i,)),
            pl.BlockSpec((BM,), lambda i: (i,)),
        ],
        out_shape=[
            jax.ShapeDtypeStruct((NP1, 128), jnp.float32),
            jax.ShapeDtypeStruct((NP1,), jnp.float32),
            jax.ShapeDtypeStruct((NP1,), jnp.float32),
            jax.ShapeDtypeStruct((NP1,), jnp.float32),
        ],
    )(agg1, g1p, dinv1, Wc1p, bc1, Wg, a_src, a_dst)


def _m3b_call(mpartT, el_self):
    BM = 2048
    grid = NP1 // BM

    def body(mp_r, els_r, out_r):
        m = jnp.max(mp_r[...], axis=1)
        out_r[...] = jnp.maximum(m, els_r[...])

    return pl.pallas_call(
        body,
        grid=(grid,),
        in_specs=[
            pl.BlockSpec((BM, 32), lambda i: (i, 0)),
            pl.BlockSpec((BM,), lambda i: (i,)),
        ],
        out_specs=pl.BlockSpec((BM,), lambda i: (i,)),
        out_shape=jax.ShapeDtypeStruct((NP1,), jnp.float32),
    )(mpartT, el_self)


def _m4_call(aggG, h2d, el_self, m_full, denpT, Wg, bg, batch1p):
    BM = 2048
    grid = NP1 // BM

    def body(agg_r, h2d_r, els_r, m_r, denp_r, wg_r, bg_r, b_r,
             pool_r, cnt_r):
        i = pl.program_id(0)
        ee_self = jnp.exp(els_r[...] - m_r[...])
        den = denp_r[:, 0] + denp_r[:, 1] + ee_self
        inv = 1.0 / (den + 1e-16)
        rows = (agg_r[...] + ee_self[:, None] * h2d_r[...]) * inv[:, None]
        h3 = jax.nn.relu(jnp.dot(rows, wg_r[...]) + bg_r[...][None, :])
        seg = b_r[...]
        mask = (seg[None, :] ==
                lax.broadcasted_iota(jnp.int32, (B, BM), 0)).astype(jnp.float32)

        @pl.when(i == 0)
        def _():
            pool_r[...] = jnp.zeros((B, 256), jnp.float32)
            cnt_r[...] = jnp.zeros((B,), jnp.float32)

        pool_r[...] += jnp.dot(mask, h3)
        cnt_r[...] += jnp.sum(mask, axis=1)

    return pl.pallas_call(
        body,
        grid=(grid,),
        in_specs=[
            pl.BlockSpec((BM, 128), lambda i: (i, 0)),
            pl.BlockSpec((BM, 128), lambda i: (i, 0)),
            pl.BlockSpec((BM,), lambda i: (i,)),
            pl.BlockSpec((BM,), lambda i: (i,)),
            pl.BlockSpec((BM, 2), lambda i: (i, 0)),
            pl.BlockSpec((128, 256), lambda i: (0, 0)),
            pl.BlockSpec((256,), lambda i: (0,)),
            pl.BlockSpec((BM,), lambda i: (i,)),
        ],
        out_specs=[
            pl.BlockSpec((B, 256), lambda i: (0, 0)),
            pl.BlockSpec((B,), lambda i: (0,)),
        ],
        out_shape=[
            jax.ShapeDtypeStruct((B, 256), jnp.float32),
            jax.ShapeDtypeStruct((B,), jnp.float32),
        ],
    )(aggG, h2d, el_self, m_full, denpT, Wg, bg, batch1p)


def _m5_call(aggs, gp1s, dinv3, bc4p, li_p, Wc5):
    BM = 1024
    grid = NP3 // BM

    def body(*refs):
        aggs = refs[0:8]
        gs = refs[8:16]
        dinv_r, bc4_r, li_r, w_r, g40_r, g41_r = refs[16:]
        dinv = dinv_r[...]
        cols = []
        for cb in range(8):
            bcol = bc4_r[...][cb * 128:(cb + 1) * 128]
            full = (aggs[cb][...] + gs[cb][...]) * dinv[:, None] + bcol[None, :]
            cols.append(jax.nn.relu(full))
        colid = lax.broadcasted_iota(jnp.int32, (BM, 128), 1)
        cols[7] = jnp.where(colid == 127, li_r[...][:, None], cols[7])
        ami = jnp.concatenate(cols, axis=1)
        p2 = jnp.dot(ami, w_r[...])
        g4 = p2 * dinv[:, None]
        g40_r[...] = g4[:, 0:128]
        g41_r[...] = g4[:, 128:256]

    nspec = pl.BlockSpec((BM, 128), lambda i: (i, 0))
    return pl.pallas_call(
        body,
        grid=(grid,),
        in_specs=[nspec] * 16 + [
            pl.BlockSpec((BM,), lambda i: (i,)),
            pl.BlockSpec((1024,), lambda i: (0,)),
            pl.BlockSpec((BM,), lambda i: (i,)),
            pl.BlockSpec((1024, 256), lambda i: (0, 0)),
        ],
        out_specs=[nspec, nspec],
        out_shape=[jax.ShapeDtypeStruct((NP3, 128), jnp.float32),
                   jax.ShapeDtypeStruct((NP3, 128), jnp.float32)],
    )(*aggs, *gp1s, dinv3, bc4p, li_p, Wc5)


def _m6_call(agg5, g4, dinv3, bc5, ami_batch_p):
    BM = 1024
    grid = NP3 // BM

    def body(a0_r, a1_r, g40_r, g41_r, dinv_r, bc5_r, b_r, pool_r, cnt_r):
        i = pl.program_id(0)
        dinv = dinv_r[...]
        agg = jnp.concatenate([a0_r[...] + g40_r[...],
                               a1_r[...] + g41_r[...]], axis=1)
        p3 = jax.nn.relu(agg * dinv[:, None] + bc5_r[...][None, :])
        seg = b_r[...]
        mask = (seg[None, :] ==
                lax.broadcasted_iota(jnp.int32, (B, BM), 0)).astype(jnp.float32)

        @pl.when(i == 0)
        def _():
            pool_r[...] = jnp.zeros((B, 256), jnp.float32)
            cnt_r[...] = jnp.zeros((B,), jnp.float32)

        pool_r[...] += jnp.dot(mask, p3)
        cnt_r[...] += jnp.sum(mask, axis=1)

    return pl.pallas_call(
        body,
        grid=(grid,),
        in_specs=[
            pl.BlockSpec((BM, 128), lambda i: (i, 0)),
            pl.BlockSpec((BM, 128), lambda i: (i, 0)),
            pl.BlockSpec((BM, 128), lambda i: (i, 0)),
            pl.BlockSpec((BM, 128), lambda i: (i, 0)),
            pl.BlockSpec((BM,), lambda i: (i,)),
            pl.BlockSpec((256,), lambda i: (0,)),
            pl.BlockSpec((BM,), lambda i: (i,)),
        ],
        out_specs=[
            pl.BlockSpec((B, 256), lambda i: (0, 0)),
            pl.BlockSpec((B,), lambda i: (0,)),
        ],
        out_shape=[
            jax.ShapeDtypeStruct((B, 256), jnp.float32),
            jax.ShapeDtypeStruct((B,), jnp.float32),
        ],
    )(agg5[0], agg5[1], g4[0], g4[1], dinv3, bc5, ami_batch_p)


def _m7_call(apool_s, cnt3, hpool_s, cnt1, Wfc1, bfc1, Wfc2, bfc2, Woutp, boutp):
    def body(ap_r, c3_r, hp_r, c1_r, w1_r, b1_r, w2_r, b2_r, wo_r, bo_r,
             out_r, h2_r):
        ap = ap_r[...] / jnp.clip(c3_r[...], 1.0, None)[:, None]
        hp = hp_r[...] / jnp.clip(c1_r[...], 1.0, None)[:, None]
        z = jnp.concatenate([ap, hp], axis=1)
        h2 = jnp.dot(z, w1_r[...]) + b1_r[...][None, :]
        x = jax.nn.relu(h2)
        x = jax.nn.relu(jnp.dot(x, w2_r[...]) + b2_r[...][None, :])
        out_r[...] = jnp.dot(x, wo_r[...]) + bo_r[...][None, :]
        h2_r[...] = h2

    return pl.pallas_call(
        body,
        out_shape=[
            jax.ShapeDtypeStruct((B, 128), jnp.float32),
            jax.ShapeDtypeStruct((B, 512), jnp.float32),
        ],
    )(apool_s, cnt3, hpool_s, cnt1, Wfc1, bfc1, Wfc2, bfc2, Woutp, boutp)


_segsum_drug = _make_sc_segsum(SP1, EP1, has_vals=False)
_segsum_prot = _make_sc_segsum(SP3, EP3, has_vals=True)
_rowagg_gcn, _NQP1 = _make_sc_rowagg(NP1, EP1, 128, 8, False)
_rowagg_gat, _ = _make_sc_rowagg(NP1, EP1, 128, 8, True)
_rowagg_p1, _NQP3 = _make_sc_rowagg(NP3, EP3, 128, 2, True)
_rowagg_p2, _ = _make_sc_rowagg(NP3, EP3, 128, 2, True)
_sc_el_drug = _make_sc_el(NP1, EP1)
_sc_max_drug = _make_sc_max(NP1, EP1)
_sc_den_drug = _make_sc_den(NP1, SP1, EP1)


def _pad2d(a, Epad, fill):
    pad = jnp.full((Epad - a.shape[0],), fill, a.dtype)
    return jnp.reshape(jnp.concatenate([a, pad]), (Epad // 128, 128))


def _padn(a, Np, fill=0):
    return jnp.concatenate(
        [a, jnp.full((Np - a.shape[0],) + a.shape[1:], fill, a.dtype)])


def _unq(agg, nq, Nq):
    return jnp.reshape(agg[:, :Nq, :], (nq * Nq, agg.shape[2]))


def kernel(x1, edge_index1, edge_attr1, batch1, x3, edge_index3, ami_dis,
           ami_batch, ami_dis_li, w1, b1, w2, b2, Wc1, bc1, Wg, a_src, a_dst,
           bg, Wc4, bc4, Wc5, bc5, Wfc1, bfc1, Wfc2, bfc2, Wout, bout):
    src1 = _pad2d(edge_index1[0], EP1, 0)
    dst1 = _pad2d(edge_index1[1], EP1, NP1)
    src3 = _pad2d(edge_index3[0], EP3, 0)
    dst3 = _pad2d(edge_index3[1], EP3, NP3)
    ew3 = _pad2d(ami_dis, EP3, 0.0)

    x1p = _padn(x1, NP1)
    batch1p = _padn(batch1, NP1, B)
    x3p = _padn(x3, NP3)
    ami_batch_p = _padn(ami_batch, NP3, B)
    li_p = _padn(ami_dis_li, NP3)

    Wc1p = jnp.pad(Wc1, ((0, 64), (0, 0)))
    Wc4p = jnp.pad(Wc4, ((0, 0), (0, 1)))
    bc4p = jnp.pad(bc4, (0, 1))
    Woutp = jnp.pad(Wout, ((0, 0), (0, 127)))
    boutp = jnp.pad(bout, (0, 127))

    degp1 = _segsum_drug(dst1)
    g1p, dinv1 = _m1_call(x1p, w1, b1, w2, b2,
                          jnp.transpose(degp1[:, :NP1]))
    agg1 = _unq(_rowagg_gcn(g1p, src1, dst1), 8, NP1 // 8)
    h2d, al_s, al_d, el_self = _m3_call(
        agg1, g1p, dinv1, Wc1p, bc1, Wg, a_src, a_dst)
    elP = _sc_el_drug(al_s, al_d, src1, dst1)
    mpart = _sc_max_drug(dst1, elP)
    m_full = _m3b_call(jnp.transpose(mpart), el_self)
    denp, eeP = _sc_den_drug(dst1, elP, m_full)
    aggG = _unq(_rowagg_gat(h2d, src1, dst1, eeP), 8, NP1 // 8)
    hpool_s, cnt1 = _m4_call(aggG, h2d, el_self, m_full,
                             jnp.transpose(denp[:, :NP1]), Wg, bg, batch1p)

    degp3 = _segsum_prot(dst3, ew3)
    m2out = _m2_call(x3p, Wc4p, jnp.transpose(degp3[:, :NP3]))
    gp1s, dinv3 = m2out[:8], m2out[8]
    aggs = [_unq(_rowagg_p1(g, src3, dst3, ew3), 2, NP3 // 2) for g in gp1s]
    g4 = _m5_call(aggs, gp1s, dinv3, bc4p, li_p, Wc5)
    agg5 = [_unq(_rowagg_p2(g, src3, dst3, ew3), 2, NP3 // 2) for g in g4]
    apool_s, cnt3 = _m6_call(agg5, g4, dinv3, bc5, ami_batch_p)

    out_full, h2 = _m7_call(apool_s, cnt3, hpool_s, cnt1,
                            Wfc1, bfc1, Wfc2, bfc2, Woutp, boutp)
    return (out_full[:, :1], h2)
